# Initial kernel scaffold; baseline (speedup 1.0000x reference)
import jax
import jax.numpy as jnp
from jax.experimental import pallas as pl


def kernel(x, edge_index, edge_attr, W_edge, b_edge, W_pre, b_pre, W_post, b_post, W_lin, b_lin):
    raise NotImplementedError("write your pallas kernel here")


# speedup vs baseline: 1.8111x; 1.8111x over previous
import functools
import numpy as np
import jax
import jax.numpy as jnp
from jax.experimental import pallas as pl
from jax.experimental.pallas import tpu as pltpu

_F = 128
_DEG_HIST_K = np.array([0, 0, 0, 0, 0, 0, 0, 0, 0, 0, 0, 0, 0, 0, 0, 0, 50, 120, 260, 480, 820, 1150, 1400, 1500, 1400, 1150, 820, 480, 260, 120, 50, 20, 20], dtype=np.float64)
_BINS_K = np.arange(_DEG_HIST_K.size, dtype=np.float64)
_AVG_DEG_LOG = float((np.log(_BINS_K + 1.0) * _DEG_HIST_K).sum() / _DEG_HIST_K.sum())

_BIG = 3.0e38


def _node_ab_body(x_ref, wd_ref, ws_ref, a_ref, b_ref):
    xb = x_ref[...]
    a_ref[...] = jnp.dot(xb, wd_ref[...], preferred_element_type=jnp.float32)
    b_ref[...] = jnp.dot(xb, ws_ref[...], preferred_element_type=jnp.float32)


def _edge_m_body(ea_ref, wc_ref, bc_ref, m_ref):
    m_ref[...] = jnp.dot(ea_ref[...], wc_ref[...],
                         preferred_element_type=jnp.float32) + bc_ref[...]


def _scatter_body(src_ref, dst_ref, m_ref, b_ref,
                  sum_ref, sq_ref, mn_ref, mx_ref, cnt_ref, eb):
    @pl.when(pl.program_id(0) == 0)
    def _init():
        zeros = jnp.zeros_like(sum_ref)
        sum_ref[...] = zeros
        sq_ref[...] = zeros
        mn_ref[...] = jnp.full_like(mn_ref, _BIG)
        mx_ref[...] = jnp.full_like(mx_ref, -_BIG)
        cnt_ref[...] = zeros

    ones = jnp.ones((1, _F), jnp.float32)

    def body(i, _):
        s = src_ref[0, 0, i]
        d = dst_ref[0, 0, i]
        u = b_ref[pl.ds(s, 1), :] + m_ref[pl.ds(i, 1), :]
        dsl = pl.ds(d, 1)
        sum_ref[dsl, :] += u
        sq_ref[dsl, :] += u * u
        mn_ref[dsl, :] = jnp.minimum(mn_ref[dsl, :], u)
        mx_ref[dsl, :] = jnp.maximum(mx_ref[dsl, :], u)
        cnt_ref[dsl, :] += ones
        return 0

    jax.lax.fori_loop(0, eb, body, 0, unroll=False)


def _post_body(x_ref, a_ref, sum_ref, sq_ref, mn_ref, mx_ref, cnt_ref,
               wx_ref, w1_ref, w2_ref, w3_ref, bf_ref, out_ref):
    deg = cnt_ref[:, 0:1]
    has = deg > 0.0
    degc = jnp.maximum(deg, 1.0)
    inv = 1.0 / degc
    a = a_ref[...]
    mean_u = sum_ref[...] * inv
    mean = jnp.where(has, a + mean_u, 0.0)
    mn = jnp.where(has, a + mn_ref[...], 0.0)
    mx = jnp.where(has, a + mx_ref[...], 0.0)
    var = jnp.maximum(sq_ref[...] * inv - mean_u * mean_u, 0.0)
    std = jnp.sqrt(jnp.where(has, var, 0.0) + 1e-5)
    agg = jnp.concatenate([mean, mn, mx, std], axis=-1)
    log_deg = jnp.log(degc + 1.0)
    amp_s = log_deg * (1.0 / _AVG_DEG_LOG)
    att_s = _AVG_DEG_LOG / log_deg
    out = jnp.dot(x_ref[...], wx_ref[...], preferred_element_type=jnp.float32)
    out += jnp.dot(agg, w1_ref[...], preferred_element_type=jnp.float32)
    out += jnp.dot(agg * amp_s, w2_ref[...], preferred_element_type=jnp.float32)
    out += jnp.dot(agg * att_s, w3_ref[...], preferred_element_type=jnp.float32)
    out_ref[...] = out + bf_ref[...]


def kernel(x, edge_index, edge_attr, W_edge, b_edge, W_pre, b_pre, W_post, b_post, W_lin, b_lin):
    n = x.shape[0]
    e = edge_index.shape[1]
    f = _F
    src = edge_index[0]
    dst = edge_index[1]

    wp_d = W_pre[:f]
    wp_s = W_pre[f:2 * f]
    wp_e = W_pre[2 * f:]
    wc = W_edge @ wp_e
    bc = (b_edge @ wp_e + b_pre)[None, :]
    w_pl = W_post @ W_lin
    wx = w_pl[:f]
    w1 = w_pl[f:5 * f]
    w2 = w_pl[5 * f:9 * f]
    w3 = w_pl[9 * f:13 * f]
    bf = (b_post @ W_lin + b_lin)[None, :]

    nb_n = 2000
    a_mat, b_mat = pl.pallas_call(
        _node_ab_body,
        grid=(n // nb_n,),
        in_specs=[
            pl.BlockSpec((nb_n, f), lambda i: (i, 0)),
            pl.BlockSpec((f, f), lambda i: (0, 0)),
            pl.BlockSpec((f, f), lambda i: (0, 0)),
        ],
        out_specs=[
            pl.BlockSpec((nb_n, f), lambda i: (i, 0)),
            pl.BlockSpec((nb_n, f), lambda i: (i, 0)),
        ],
        out_shape=[jax.ShapeDtypeStruct((n, f), jnp.float32)] * 2,
    )(x, wp_d, wp_s)

    eb_m = 4000
    m_mat = pl.pallas_call(
        _edge_m_body,
        grid=(e // eb_m,),
        in_specs=[
            pl.BlockSpec((eb_m, f), lambda i: (i, 0)),
            pl.BlockSpec((f, f), lambda i: (0, 0)),
            pl.BlockSpec((1, f), lambda i: (0, 0)),
        ],
        out_specs=pl.BlockSpec((eb_m, f), lambda i: (i, 0)),
        out_shape=jax.ShapeDtypeStruct((e, f), jnp.float32),
    )(edge_attr, wc, bc)

    eb = 2560
    n_eb = e // eb
    src3 = src.reshape(n_eb, 1, eb)
    dst3 = dst.reshape(n_eb, 1, eb)
    acc_shape = jax.ShapeDtypeStruct((n, f), jnp.float32)
    s_u, s_uu, mn_u, mx_u, cnt = pl.pallas_call(
        functools.partial(_scatter_body, eb=eb),
        grid=(n_eb,),
        in_specs=[
            pl.BlockSpec((1, 1, eb), lambda i: (i, 0, 0), memory_space=pltpu.SMEM),
            pl.BlockSpec((1, 1, eb), lambda i: (i, 0, 0), memory_space=pltpu.SMEM),
            pl.BlockSpec((eb, f), lambda i: (i, 0)),
            pl.BlockSpec((n, f), lambda i: (0, 0)),
        ],
        out_specs=[pl.BlockSpec((n, f), lambda i: (0, 0))] * 5,
        out_shape=[acc_shape] * 5,
    )(src3, dst3, m_mat, b_mat)

    out = pl.pallas_call(
        _post_body,
        grid=(n // nb_n,),
        in_specs=[
            pl.BlockSpec((nb_n, f), lambda i: (i, 0)),
            pl.BlockSpec((nb_n, f), lambda i: (i, 0)),
            pl.BlockSpec((nb_n, f), lambda i: (i, 0)),
            pl.BlockSpec((nb_n, f), lambda i: (i, 0)),
            pl.BlockSpec((nb_n, f), lambda i: (i, 0)),
            pl.BlockSpec((nb_n, f), lambda i: (i, 0)),
            pl.BlockSpec((nb_n, f), lambda i: (i, 0)),
            pl.BlockSpec((f, f), lambda i: (0, 0)),
            pl.BlockSpec((4 * f, f), lambda i: (0, 0)),
            pl.BlockSpec((4 * f, f), lambda i: (0, 0)),
            pl.BlockSpec((4 * f, f), lambda i: (0, 0)),
            pl.BlockSpec((1, f), lambda i: (0, 0)),
        ],
        out_specs=pl.BlockSpec((nb_n, f), lambda i: (i, 0)),
        out_shape=jax.ShapeDtypeStruct((n, f), jnp.float32),
    )(x, a_mat, s_u, s_uu, mn_u, mx_u, cnt, wx, w1, w2, w3, bf)
    return out


# speedup vs baseline: 2.4022x; 1.3264x over previous
import functools
import numpy as np
import jax
import jax.numpy as jnp
from jax.experimental import pallas as pl
from jax.experimental.pallas import tpu as pltpu

_F = 128
_DEG_HIST_K = np.array([0, 0, 0, 0, 0, 0, 0, 0, 0, 0, 0, 0, 0, 0, 0, 0, 50, 120, 260, 480, 820, 1150, 1400, 1500, 1400, 1150, 820, 480, 260, 120, 50, 20, 20], dtype=np.float64)
_BINS_K = np.arange(_DEG_HIST_K.size, dtype=np.float64)
_AVG_DEG_LOG = float((np.log(_BINS_K + 1.0) * _DEG_HIST_K).sum() / _DEG_HIST_K.sum())

_BIG = 3.0e38


def _node_ab_body(x_ref, wd_ref, ws_ref, a_ref, b_ref):
    xb = x_ref[...]
    a_ref[...] = jnp.dot(xb, wd_ref[...], preferred_element_type=jnp.float32)
    b_ref[...] = jnp.dot(xb, ws_ref[...], preferred_element_type=jnp.float32)


def _edge_m_body(ea_ref, wc_ref, bc_ref, m_ref):
    m_ref[...] = jnp.dot(ea_ref[...], wc_ref[...],
                         preferred_element_type=jnp.float32) + bc_ref[...]


def _scatter_body(src_ref, dst_ref, ea_ref, wc_ref, bc_ref, b_ref,
                  sum_ref, sq_ref, mn_ref, mx_ref, cnt_ref, m_scr, eb):
    @pl.when(pl.program_id(0) == 0)
    def _init():
        zeros = jnp.zeros_like(sum_ref)
        sum_ref[...] = zeros
        sq_ref[...] = zeros
        mn_ref[...] = jnp.full_like(mn_ref, _BIG)
        mx_ref[...] = jnp.full_like(mx_ref, -_BIG)
        cnt_ref[...] = zeros

    m_scr[...] = jnp.dot(ea_ref[...], wc_ref[...],
                         preferred_element_type=jnp.float32) + bc_ref[...]

    ones = jnp.ones((1, _F), jnp.float32)

    def body(i, _):
        s = src_ref[0, 0, i]
        d = dst_ref[0, 0, i]
        u = b_ref[pl.ds(s, 1), :] + m_scr[pl.ds(i, 1), :]
        dsl = pl.ds(d, 1)
        sum_ref[dsl, :] += u
        sq_ref[dsl, :] += u * u
        mn_ref[dsl, :] = jnp.minimum(mn_ref[dsl, :], u)
        mx_ref[dsl, :] = jnp.maximum(mx_ref[dsl, :], u)
        cnt_ref[dsl, :] += ones
        return 0

    jax.lax.fori_loop(0, eb, body, 0, unroll=2)


def _post_body(x_ref, a_ref, sum_ref, sq_ref, mn_ref, mx_ref, cnt_ref,
               wx_ref, w1_ref, w2_ref, w3_ref, bf_ref, out_ref):
    deg = cnt_ref[:, 0:1]
    has = deg > 0.0
    degc = jnp.maximum(deg, 1.0)
    inv = 1.0 / degc
    a = a_ref[...]
    mean_u = sum_ref[...] * inv
    mean = jnp.where(has, a + mean_u, 0.0)
    mn = jnp.where(has, a + mn_ref[...], 0.0)
    mx = jnp.where(has, a + mx_ref[...], 0.0)
    var = jnp.maximum(sq_ref[...] * inv - mean_u * mean_u, 0.0)
    std = jnp.sqrt(jnp.where(has, var, 0.0) + 1e-5)
    agg = jnp.concatenate([mean, mn, mx, std], axis=-1)
    log_deg = jnp.log(degc + 1.0)
    amp_s = log_deg * (1.0 / _AVG_DEG_LOG)
    att_s = _AVG_DEG_LOG / log_deg
    out = jnp.dot(x_ref[...], wx_ref[...], preferred_element_type=jnp.float32)
    out += jnp.dot(agg, w1_ref[...], preferred_element_type=jnp.float32)
    out += jnp.dot(agg * amp_s, w2_ref[...], preferred_element_type=jnp.float32)
    out += jnp.dot(agg * att_s, w3_ref[...], preferred_element_type=jnp.float32)
    out_ref[...] = out + bf_ref[...]


def kernel(x, edge_index, edge_attr, W_edge, b_edge, W_pre, b_pre, W_post, b_post, W_lin, b_lin):
    n = x.shape[0]
    e = edge_index.shape[1]
    f = _F
    src = edge_index[0]
    dst = edge_index[1]

    wp_d = W_pre[:f]
    wp_s = W_pre[f:2 * f]
    wp_e = W_pre[2 * f:]
    wc = W_edge @ wp_e
    bc = (b_edge @ wp_e + b_pre)[None, :]
    w_pl = W_post @ W_lin
    wx = w_pl[:f]
    w1 = w_pl[f:5 * f]
    w2 = w_pl[5 * f:9 * f]
    w3 = w_pl[9 * f:13 * f]
    bf = (b_post @ W_lin + b_lin)[None, :]

    nb_n = 2000
    a_mat, b_mat = pl.pallas_call(
        _node_ab_body,
        grid=(n // nb_n,),
        in_specs=[
            pl.BlockSpec((nb_n, f), lambda i: (i, 0)),
            pl.BlockSpec((f, f), lambda i: (0, 0)),
            pl.BlockSpec((f, f), lambda i: (0, 0)),
        ],
        out_specs=[
            pl.BlockSpec((nb_n, f), lambda i: (i, 0)),
            pl.BlockSpec((nb_n, f), lambda i: (i, 0)),
        ],
        out_shape=[jax.ShapeDtypeStruct((n, f), jnp.float32)] * 2,
    )(x, wp_d, wp_s)

    eb = 2560
    n_eb = e // eb
    src3 = src.reshape(n_eb, 1, eb)
    dst3 = dst.reshape(n_eb, 1, eb)
    acc_shape = jax.ShapeDtypeStruct((n, f), jnp.float32)
    s_u, s_uu, mn_u, mx_u, cnt = pl.pallas_call(
        functools.partial(_scatter_body, eb=eb),
        grid=(n_eb,),
        in_specs=[
            pl.BlockSpec((1, 1, eb), lambda i: (i, 0, 0), memory_space=pltpu.SMEM),
            pl.BlockSpec((1, 1, eb), lambda i: (i, 0, 0), memory_space=pltpu.SMEM),
            pl.BlockSpec((eb, f), lambda i: (i, 0)),
            pl.BlockSpec((f, f), lambda i: (0, 0)),
            pl.BlockSpec((1, f), lambda i: (0, 0)),
            pl.BlockSpec((n, f), lambda i: (0, 0)),
        ],
        out_specs=[pl.BlockSpec((n, f), lambda i: (0, 0))] * 5,
        out_shape=[acc_shape] * 5,
        scratch_shapes=[pltpu.VMEM((eb, f), jnp.float32)],
    )(src3, dst3, edge_attr, wc, bc, b_mat)

    out = pl.pallas_call(
        _post_body,
        grid=(n // nb_n,),
        in_specs=[
            pl.BlockSpec((nb_n, f), lambda i: (i, 0)),
            pl.BlockSpec((nb_n, f), lambda i: (i, 0)),
            pl.BlockSpec((nb_n, f), lambda i: (i, 0)),
            pl.BlockSpec((nb_n, f), lambda i: (i, 0)),
            pl.BlockSpec((nb_n, f), lambda i: (i, 0)),
            pl.BlockSpec((nb_n, f), lambda i: (i, 0)),
            pl.BlockSpec((nb_n, f), lambda i: (i, 0)),
            pl.BlockSpec((f, f), lambda i: (0, 0)),
            pl.BlockSpec((4 * f, f), lambda i: (0, 0)),
            pl.BlockSpec((4 * f, f), lambda i: (0, 0)),
            pl.BlockSpec((4 * f, f), lambda i: (0, 0)),
            pl.BlockSpec((1, f), lambda i: (0, 0)),
        ],
        out_specs=pl.BlockSpec((nb_n, f), lambda i: (i, 0)),
        out_shape=jax.ShapeDtypeStruct((n, f), jnp.float32),
    )(x, a_mat, s_u, s_uu, mn_u, mx_u, cnt, wx, w1, w2, w3, bf)
    return out


# speedup vs baseline: 2.7564x; 1.1474x over previous
import functools
import numpy as np
import jax
import jax.numpy as jnp
from jax.experimental import pallas as pl
from jax.experimental.pallas import tpu as pltpu

_F = 128
_DEG_HIST_K = np.array([0, 0, 0, 0, 0, 0, 0, 0, 0, 0, 0, 0, 0, 0, 0, 0, 50, 120, 260, 480, 820, 1150, 1400, 1500, 1400, 1150, 820, 480, 260, 120, 50, 20, 20], dtype=np.float64)
_BINS_K = np.arange(_DEG_HIST_K.size, dtype=np.float64)
_AVG_DEG_LOG = float((np.log(_BINS_K + 1.0) * _DEG_HIST_K).sum() / _DEG_HIST_K.sum())

_BIG = 3.0e38


def _node_ab_body(x_ref, wd_ref, ws_ref, a_ref, b_ref):
    xb = x_ref[...]
    a_ref[...] = jnp.dot(xb, wd_ref[...], preferred_element_type=jnp.float32)
    b_ref[...] = jnp.dot(xb, ws_ref[...], preferred_element_type=jnp.float32)


def _edge_m_body(ea_ref, wc_ref, bc_ref, m_ref):
    m_ref[...] = jnp.dot(ea_ref[...], wc_ref[...],
                         preferred_element_type=jnp.float32) + bc_ref[...]


def _scatter_body(src_ref, dst_ref, ea_ref, wc_ref, bc_ref, b_ref,
                  sum_ref, sq_ref, mn_ref, mx_ref, cnt_ref, m_scr, eb):
    @pl.when(pl.program_id(0) == 0)
    def _init():
        zeros = jnp.zeros_like(sum_ref)
        sum_ref[...] = zeros
        sq_ref[...] = zeros
        mn_ref[...] = jnp.full_like(mn_ref, _BIG)
        mx_ref[...] = jnp.full_like(mx_ref, -_BIG)
        cnt_ref[...] = zeros

    m_scr[...] = jnp.dot(ea_ref[...], wc_ref[...],
                         preferred_element_type=jnp.float32) + bc_ref[...]

    ones = jnp.ones((1, _F), jnp.float32)

    def body(i, _):
        s = src_ref[0, 0, i]
        d = dst_ref[0, 0, i]
        u = b_ref[pl.ds(s, 1), :] + m_scr[pl.ds(i, 1), :]
        dsl = pl.ds(d, 1)
        sum_ref[dsl, :] += u
        sq_ref[dsl, :] += u * u
        mn_ref[dsl, :] = jnp.minimum(mn_ref[dsl, :], u)
        mx_ref[dsl, :] = jnp.maximum(mx_ref[dsl, :], u)
        cnt_ref[dsl, :] += ones
        return 0

    jax.lax.fori_loop(0, eb, body, 0, unroll=4)


def _post_body(x_ref, a_ref, sum_ref, sq_ref, mn_ref, mx_ref, cnt_ref,
               wx_ref, w1_ref, w2_ref, w3_ref, bf_ref, out_ref):
    deg = cnt_ref[:, 0:1]
    has = deg > 0.0
    degc = jnp.maximum(deg, 1.0)
    inv = 1.0 / degc
    a = a_ref[...]
    mean_u = sum_ref[...] * inv
    mean = jnp.where(has, a + mean_u, 0.0)
    mn = jnp.where(has, a + mn_ref[...], 0.0)
    mx = jnp.where(has, a + mx_ref[...], 0.0)
    var = jnp.maximum(sq_ref[...] * inv - mean_u * mean_u, 0.0)
    std = jnp.sqrt(jnp.where(has, var, 0.0) + 1e-5)
    agg = jnp.concatenate([mean, mn, mx, std], axis=-1)
    log_deg = jnp.log(degc + 1.0)
    amp_s = log_deg * (1.0 / _AVG_DEG_LOG)
    att_s = _AVG_DEG_LOG / log_deg
    out = jnp.dot(x_ref[...], wx_ref[...], preferred_element_type=jnp.float32)
    out += jnp.dot(agg, w1_ref[...], preferred_element_type=jnp.float32)
    out += jnp.dot(agg * amp_s, w2_ref[...], preferred_element_type=jnp.float32)
    out += jnp.dot(agg * att_s, w3_ref[...], preferred_element_type=jnp.float32)
    out_ref[...] = out + bf_ref[...]


def kernel(x, edge_index, edge_attr, W_edge, b_edge, W_pre, b_pre, W_post, b_post, W_lin, b_lin):
    n = x.shape[0]
    e = edge_index.shape[1]
    f = _F
    src = edge_index[0]
    dst = edge_index[1]

    wp_d = W_pre[:f]
    wp_s = W_pre[f:2 * f]
    wp_e = W_pre[2 * f:]
    wc = W_edge @ wp_e
    bc = (b_edge @ wp_e + b_pre)[None, :]
    w_pl = W_post @ W_lin
    wx = w_pl[:f]
    w1 = w_pl[f:5 * f]
    w2 = w_pl[5 * f:9 * f]
    w3 = w_pl[9 * f:13 * f]
    bf = (b_post @ W_lin + b_lin)[None, :]

    nb_n = 2000
    a_mat, b_mat = pl.pallas_call(
        _node_ab_body,
        grid=(n // nb_n,),
        in_specs=[
            pl.BlockSpec((nb_n, f), lambda i: (i, 0)),
            pl.BlockSpec((f, f), lambda i: (0, 0)),
            pl.BlockSpec((f, f), lambda i: (0, 0)),
        ],
        out_specs=[
            pl.BlockSpec((nb_n, f), lambda i: (i, 0)),
            pl.BlockSpec((nb_n, f), lambda i: (i, 0)),
        ],
        out_shape=[jax.ShapeDtypeStruct((n, f), jnp.float32)] * 2,
    )(x, wp_d, wp_s)

    eb = 2560
    n_eb = e // eb
    src3 = src.reshape(n_eb, 1, eb)
    dst3 = dst.reshape(n_eb, 1, eb)
    acc_shape = jax.ShapeDtypeStruct((n, f), jnp.float32)
    s_u, s_uu, mn_u, mx_u, cnt = pl.pallas_call(
        functools.partial(_scatter_body, eb=eb),
        grid=(n_eb,),
        in_specs=[
            pl.BlockSpec((1, 1, eb), lambda i: (i, 0, 0), memory_space=pltpu.SMEM),
            pl.BlockSpec((1, 1, eb), lambda i: (i, 0, 0), memory_space=pltpu.SMEM),
            pl.BlockSpec((eb, f), lambda i: (i, 0)),
            pl.BlockSpec((f, f), lambda i: (0, 0)),
            pl.BlockSpec((1, f), lambda i: (0, 0)),
            pl.BlockSpec((n, f), lambda i: (0, 0)),
        ],
        out_specs=[pl.BlockSpec((n, f), lambda i: (0, 0))] * 5,
        out_shape=[acc_shape] * 5,
        scratch_shapes=[pltpu.VMEM((eb, f), jnp.float32)],
    )(src3, dst3, edge_attr, wc, bc, b_mat)

    out = pl.pallas_call(
        _post_body,
        grid=(n // nb_n,),
        in_specs=[
            pl.BlockSpec((nb_n, f), lambda i: (i, 0)),
            pl.BlockSpec((nb_n, f), lambda i: (i, 0)),
            pl.BlockSpec((nb_n, f), lambda i: (i, 0)),
            pl.BlockSpec((nb_n, f), lambda i: (i, 0)),
            pl.BlockSpec((nb_n, f), lambda i: (i, 0)),
            pl.BlockSpec((nb_n, f), lambda i: (i, 0)),
            pl.BlockSpec((nb_n, f), lambda i: (i, 0)),
            pl.BlockSpec((f, f), lambda i: (0, 0)),
            pl.BlockSpec((4 * f, f), lambda i: (0, 0)),
            pl.BlockSpec((4 * f, f), lambda i: (0, 0)),
            pl.BlockSpec((4 * f, f), lambda i: (0, 0)),
            pl.BlockSpec((1, f), lambda i: (0, 0)),
        ],
        out_specs=pl.BlockSpec((nb_n, f), lambda i: (i, 0)),
        out_shape=jax.ShapeDtypeStruct((n, f), jnp.float32),
    )(x, a_mat, s_u, s_uu, mn_u, mx_u, cnt, wx, w1, w2, w3, bf)
    return out


# speedup vs baseline: 2.9103x; 1.0558x over previous
import functools
import numpy as np
import jax
import jax.numpy as jnp
from jax.experimental import pallas as pl
from jax.experimental.pallas import tpu as pltpu

_F = 128
_DEG_HIST_K = np.array([0, 0, 0, 0, 0, 0, 0, 0, 0, 0, 0, 0, 0, 0, 0, 0, 50, 120, 260, 480, 820, 1150, 1400, 1500, 1400, 1150, 820, 480, 260, 120, 50, 20, 20], dtype=np.float64)
_BINS_K = np.arange(_DEG_HIST_K.size, dtype=np.float64)
_AVG_DEG_LOG = float((np.log(_BINS_K + 1.0) * _DEG_HIST_K).sum() / _DEG_HIST_K.sum())

_BIG = 3.0e38


def _node_ab_body(x_ref, wd_ref, ws_ref, a_ref, b_ref):
    xb = x_ref[...]
    a_ref[...] = jnp.dot(xb, wd_ref[...], preferred_element_type=jnp.float32)
    b_ref[...] = jnp.dot(xb, ws_ref[...], preferred_element_type=jnp.float32)


def _edge_m_body(ea_ref, wc_ref, bc_ref, m_ref):
    m_ref[...] = jnp.dot(ea_ref[...], wc_ref[...],
                         preferred_element_type=jnp.float32) + bc_ref[...]


def _scatter_body(src_ref, dst_ref, ea_ref, wc_ref, bc_ref, b_ref,
                  sum_ref, sq_ref, mn_ref, mx_ref, cnt_ref, m_scr, eb):
    @pl.when(pl.program_id(0) == 0)
    def _init():
        zeros = jnp.zeros_like(sum_ref)
        sum_ref[...] = zeros
        sq_ref[...] = zeros
        mn_ref[...] = jnp.full_like(mn_ref, _BIG)
        mx_ref[...] = jnp.full_like(mx_ref, -_BIG)
        cnt_ref[...] = zeros

    m_scr[...] = jnp.dot(ea_ref[...], wc_ref[...],
                         preferred_element_type=jnp.float32) + bc_ref[...]

    ones = jnp.ones((1, _F), jnp.float32)

    def body(i, _):
        s = src_ref[0, 0, i]
        d = dst_ref[0, 0, i]
        u = b_ref[pl.ds(s, 1), :] + m_scr[pl.ds(i, 1), :]
        dsl = pl.ds(d, 1)
        sum_ref[dsl, :] += u
        sq_ref[dsl, :] += u * u
        mn_ref[dsl, :] = jnp.minimum(mn_ref[dsl, :], u)
        mx_ref[dsl, :] = jnp.maximum(mx_ref[dsl, :], u)
        cnt_ref[dsl, :] += ones
        return 0

    jax.lax.fori_loop(0, eb, body, 0, unroll=8)


def _post_body(x_ref, a_ref, sum_ref, sq_ref, mn_ref, mx_ref, cnt_ref,
               wx_ref, w1_ref, w2_ref, w3_ref, bf_ref, out_ref):
    deg = cnt_ref[:, 0:1]
    has = deg > 0.0
    degc = jnp.maximum(deg, 1.0)
    inv = 1.0 / degc
    a = a_ref[...]
    mean_u = sum_ref[...] * inv
    mean = jnp.where(has, a + mean_u, 0.0)
    mn = jnp.where(has, a + mn_ref[...], 0.0)
    mx = jnp.where(has, a + mx_ref[...], 0.0)
    var = jnp.maximum(sq_ref[...] * inv - mean_u * mean_u, 0.0)
    std = jnp.sqrt(jnp.where(has, var, 0.0) + 1e-5)
    agg = jnp.concatenate([mean, mn, mx, std], axis=-1)
    log_deg = jnp.log(degc + 1.0)
    amp_s = log_deg * (1.0 / _AVG_DEG_LOG)
    att_s = _AVG_DEG_LOG / log_deg
    out = jnp.dot(x_ref[...], wx_ref[...], preferred_element_type=jnp.float32)
    out += jnp.dot(agg, w1_ref[...], preferred_element_type=jnp.float32)
    out += jnp.dot(agg * amp_s, w2_ref[...], preferred_element_type=jnp.float32)
    out += jnp.dot(agg * att_s, w3_ref[...], preferred_element_type=jnp.float32)
    out_ref[...] = out + bf_ref[...]


def kernel(x, edge_index, edge_attr, W_edge, b_edge, W_pre, b_pre, W_post, b_post, W_lin, b_lin):
    n = x.shape[0]
    e = edge_index.shape[1]
    f = _F
    src = edge_index[0]
    dst = edge_index[1]

    wp_d = W_pre[:f]
    wp_s = W_pre[f:2 * f]
    wp_e = W_pre[2 * f:]
    wc = W_edge @ wp_e
    bc = (b_edge @ wp_e + b_pre)[None, :]
    w_pl = W_post @ W_lin
    wx = w_pl[:f]
    w1 = w_pl[f:5 * f]
    w2 = w_pl[5 * f:9 * f]
    w3 = w_pl[9 * f:13 * f]
    bf = (b_post @ W_lin + b_lin)[None, :]

    nb_n = 2000
    a_mat, b_mat = pl.pallas_call(
        _node_ab_body,
        grid=(n // nb_n,),
        in_specs=[
            pl.BlockSpec((nb_n, f), lambda i: (i, 0)),
            pl.BlockSpec((f, f), lambda i: (0, 0)),
            pl.BlockSpec((f, f), lambda i: (0, 0)),
        ],
        out_specs=[
            pl.BlockSpec((nb_n, f), lambda i: (i, 0)),
            pl.BlockSpec((nb_n, f), lambda i: (i, 0)),
        ],
        out_shape=[jax.ShapeDtypeStruct((n, f), jnp.float32)] * 2,
    )(x, wp_d, wp_s)

    eb = 2560
    n_eb = e // eb
    src3 = src.reshape(n_eb, 1, eb)
    dst3 = dst.reshape(n_eb, 1, eb)
    acc_shape = jax.ShapeDtypeStruct((n, f), jnp.float32)
    s_u, s_uu, mn_u, mx_u, cnt = pl.pallas_call(
        functools.partial(_scatter_body, eb=eb),
        grid=(n_eb,),
        in_specs=[
            pl.BlockSpec((1, 1, eb), lambda i: (i, 0, 0), memory_space=pltpu.SMEM),
            pl.BlockSpec((1, 1, eb), lambda i: (i, 0, 0), memory_space=pltpu.SMEM),
            pl.BlockSpec((eb, f), lambda i: (i, 0)),
            pl.BlockSpec((f, f), lambda i: (0, 0)),
            pl.BlockSpec((1, f), lambda i: (0, 0)),
            pl.BlockSpec((n, f), lambda i: (0, 0)),
        ],
        out_specs=[pl.BlockSpec((n, f), lambda i: (0, 0))] * 5,
        out_shape=[acc_shape] * 5,
        scratch_shapes=[pltpu.VMEM((eb, f), jnp.float32)],
    )(src3, dst3, edge_attr, wc, bc, b_mat)

    out = pl.pallas_call(
        _post_body,
        grid=(n // nb_n,),
        in_specs=[
            pl.BlockSpec((nb_n, f), lambda i: (i, 0)),
            pl.BlockSpec((nb_n, f), lambda i: (i, 0)),
            pl.BlockSpec((nb_n, f), lambda i: (i, 0)),
            pl.BlockSpec((nb_n, f), lambda i: (i, 0)),
            pl.BlockSpec((nb_n, f), lambda i: (i, 0)),
            pl.BlockSpec((nb_n, f), lambda i: (i, 0)),
            pl.BlockSpec((nb_n, f), lambda i: (i, 0)),
            pl.BlockSpec((f, f), lambda i: (0, 0)),
            pl.BlockSpec((4 * f, f), lambda i: (0, 0)),
            pl.BlockSpec((4 * f, f), lambda i: (0, 0)),
            pl.BlockSpec((4 * f, f), lambda i: (0, 0)),
            pl.BlockSpec((1, f), lambda i: (0, 0)),
        ],
        out_specs=pl.BlockSpec((nb_n, f), lambda i: (i, 0)),
        out_shape=jax.ShapeDtypeStruct((n, f), jnp.float32),
    )(x, a_mat, s_u, s_uu, mn_u, mx_u, cnt, wx, w1, w2, w3, bf)
    return out


# speedup vs baseline: 2.9926x; 1.0283x over previous
import functools
import numpy as np
import jax
import jax.numpy as jnp
from jax.experimental import pallas as pl
from jax.experimental.pallas import tpu as pltpu

_F = 128
_DEG_HIST_K = np.array([0, 0, 0, 0, 0, 0, 0, 0, 0, 0, 0, 0, 0, 0, 0, 0, 50, 120, 260, 480, 820, 1150, 1400, 1500, 1400, 1150, 820, 480, 260, 120, 50, 20, 20], dtype=np.float64)
_BINS_K = np.arange(_DEG_HIST_K.size, dtype=np.float64)
_AVG_DEG_LOG = float((np.log(_BINS_K + 1.0) * _DEG_HIST_K).sum() / _DEG_HIST_K.sum())

_BIG = 3.0e38


def _node_ab_body(x_ref, wd_ref, ws_ref, a_ref, b_ref):
    xb = x_ref[...]
    a_ref[...] = jnp.dot(xb, wd_ref[...], preferred_element_type=jnp.float32)
    b_ref[...] = jnp.dot(xb, ws_ref[...], preferred_element_type=jnp.float32)


def _edge_m_body(ea_ref, wc_ref, bc_ref, m_ref):
    m_ref[...] = jnp.dot(ea_ref[...], wc_ref[...],
                         preferred_element_type=jnp.float32) + bc_ref[...]


def _scatter_body(src_ref, dst_ref, ea_ref, wc_ref, bc_ref, b_ref,
                  sum_ref, sq_ref, mn_ref, mx_ref, cnt_ref, m_scr, eb):
    @pl.when(pl.program_id(0) == 0)
    def _init():
        zeros = jnp.zeros_like(sum_ref)
        sum_ref[...] = zeros
        sq_ref[...] = zeros
        mn_ref[...] = jnp.full_like(mn_ref, _BIG)
        mx_ref[...] = jnp.full_like(mx_ref, -_BIG)
        cnt_ref[...] = zeros

    m_scr[...] = jnp.dot(ea_ref[...], wc_ref[...],
                         preferred_element_type=jnp.float32) + bc_ref[...]

    ones = jnp.ones((1, _F), jnp.float32)

    def body(i, _):
        s = src_ref[0, 0, i]
        d = dst_ref[0, 0, i]
        u = b_ref[pl.ds(s, 1), :] + m_scr[pl.ds(i, 1), :]
        dsl = pl.ds(d, 1)
        sum_ref[dsl, :] += u
        sq_ref[dsl, :] += u * u
        mn_ref[dsl, :] = jnp.minimum(mn_ref[dsl, :], u)
        mx_ref[dsl, :] = jnp.maximum(mx_ref[dsl, :], u)
        cnt_ref[dsl, :] += ones
        return 0

    jax.lax.fori_loop(0, eb, body, 0, unroll=16)


def _post_body(x_ref, a_ref, sum_ref, sq_ref, mn_ref, mx_ref, cnt_ref,
               wx_ref, w1_ref, w2_ref, w3_ref, bf_ref, out_ref):
    deg = cnt_ref[:, 0:1]
    has = deg > 0.0
    degc = jnp.maximum(deg, 1.0)
    inv = 1.0 / degc
    a = a_ref[...]
    mean_u = sum_ref[...] * inv
    mean = jnp.where(has, a + mean_u, 0.0)
    mn = jnp.where(has, a + mn_ref[...], 0.0)
    mx = jnp.where(has, a + mx_ref[...], 0.0)
    var = jnp.maximum(sq_ref[...] * inv - mean_u * mean_u, 0.0)
    std = jnp.sqrt(jnp.where(has, var, 0.0) + 1e-5)
    agg = jnp.concatenate([mean, mn, mx, std], axis=-1)
    log_deg = jnp.log(degc + 1.0)
    amp_s = log_deg * (1.0 / _AVG_DEG_LOG)
    att_s = _AVG_DEG_LOG / log_deg
    out = jnp.dot(x_ref[...], wx_ref[...], preferred_element_type=jnp.float32)
    out += jnp.dot(agg, w1_ref[...], preferred_element_type=jnp.float32)
    out += jnp.dot(agg * amp_s, w2_ref[...], preferred_element_type=jnp.float32)
    out += jnp.dot(agg * att_s, w3_ref[...], preferred_element_type=jnp.float32)
    out_ref[...] = out + bf_ref[...]


def kernel(x, edge_index, edge_attr, W_edge, b_edge, W_pre, b_pre, W_post, b_post, W_lin, b_lin):
    n = x.shape[0]
    e = edge_index.shape[1]
    f = _F
    src = edge_index[0]
    dst = edge_index[1]

    wp_d = W_pre[:f]
    wp_s = W_pre[f:2 * f]
    wp_e = W_pre[2 * f:]
    wc = W_edge @ wp_e
    bc = (b_edge @ wp_e + b_pre)[None, :]
    w_pl = W_post @ W_lin
    wx = w_pl[:f]
    w1 = w_pl[f:5 * f]
    w2 = w_pl[5 * f:9 * f]
    w3 = w_pl[9 * f:13 * f]
    bf = (b_post @ W_lin + b_lin)[None, :]

    nb_n = 2000
    a_mat, b_mat = pl.pallas_call(
        _node_ab_body,
        grid=(n // nb_n,),
        in_specs=[
            pl.BlockSpec((nb_n, f), lambda i: (i, 0)),
            pl.BlockSpec((f, f), lambda i: (0, 0)),
            pl.BlockSpec((f, f), lambda i: (0, 0)),
        ],
        out_specs=[
            pl.BlockSpec((nb_n, f), lambda i: (i, 0)),
            pl.BlockSpec((nb_n, f), lambda i: (i, 0)),
        ],
        out_shape=[jax.ShapeDtypeStruct((n, f), jnp.float32)] * 2,
    )(x, wp_d, wp_s)

    eb = 2560
    n_eb = e // eb
    src3 = src.reshape(n_eb, 1, eb)
    dst3 = dst.reshape(n_eb, 1, eb)
    acc_shape = jax.ShapeDtypeStruct((n, f), jnp.float32)
    s_u, s_uu, mn_u, mx_u, cnt = pl.pallas_call(
        functools.partial(_scatter_body, eb=eb),
        grid=(n_eb,),
        in_specs=[
            pl.BlockSpec((1, 1, eb), lambda i: (i, 0, 0), memory_space=pltpu.SMEM),
            pl.BlockSpec((1, 1, eb), lambda i: (i, 0, 0), memory_space=pltpu.SMEM),
            pl.BlockSpec((eb, f), lambda i: (i, 0)),
            pl.BlockSpec((f, f), lambda i: (0, 0)),
            pl.BlockSpec((1, f), lambda i: (0, 0)),
            pl.BlockSpec((n, f), lambda i: (0, 0)),
        ],
        out_specs=[pl.BlockSpec((n, f), lambda i: (0, 0))] * 5,
        out_shape=[acc_shape] * 5,
        scratch_shapes=[pltpu.VMEM((eb, f), jnp.float32)],
    )(src3, dst3, edge_attr, wc, bc, b_mat)

    out = pl.pallas_call(
        _post_body,
        grid=(n // nb_n,),
        in_specs=[
            pl.BlockSpec((nb_n, f), lambda i: (i, 0)),
            pl.BlockSpec((nb_n, f), lambda i: (i, 0)),
            pl.BlockSpec((nb_n, f), lambda i: (i, 0)),
            pl.BlockSpec((nb_n, f), lambda i: (i, 0)),
            pl.BlockSpec((nb_n, f), lambda i: (i, 0)),
            pl.BlockSpec((nb_n, f), lambda i: (i, 0)),
            pl.BlockSpec((nb_n, f), lambda i: (i, 0)),
            pl.BlockSpec((f, f), lambda i: (0, 0)),
            pl.BlockSpec((4 * f, f), lambda i: (0, 0)),
            pl.BlockSpec((4 * f, f), lambda i: (0, 0)),
            pl.BlockSpec((4 * f, f), lambda i: (0, 0)),
            pl.BlockSpec((1, f), lambda i: (0, 0)),
        ],
        out_specs=pl.BlockSpec((nb_n, f), lambda i: (i, 0)),
        out_shape=jax.ShapeDtypeStruct((n, f), jnp.float32),
    )(x, a_mat, s_u, s_uu, mn_u, mx_u, cnt, wx, w1, w2, w3, bf)
    return out


# speedup vs baseline: 3.0363x; 1.0146x over previous
import functools
import numpy as np
import jax
import jax.numpy as jnp
from jax.experimental import pallas as pl
from jax.experimental.pallas import tpu as pltpu

_F = 128
_DEG_HIST_K = np.array([0, 0, 0, 0, 0, 0, 0, 0, 0, 0, 0, 0, 0, 0, 0, 0, 50, 120, 260, 480, 820, 1150, 1400, 1500, 1400, 1150, 820, 480, 260, 120, 50, 20, 20], dtype=np.float64)
_BINS_K = np.arange(_DEG_HIST_K.size, dtype=np.float64)
_AVG_DEG_LOG = float((np.log(_BINS_K + 1.0) * _DEG_HIST_K).sum() / _DEG_HIST_K.sum())

_BIG = 3.0e38


def _node_ab_body(x_ref, wd_ref, ws_ref, a_ref, b_ref):
    xb = x_ref[...]
    a_ref[...] = jnp.dot(xb, wd_ref[...], preferred_element_type=jnp.float32)
    b_ref[...] = jnp.dot(xb, ws_ref[...], preferred_element_type=jnp.float32)


def _scatter_body(src_ref, dst_ref, ea_ref, wc_ref, bc_ref, b_ref,
                  sum_ref, sq_ref, mn_ref, mx_ref, cnt_ref, m_scr, eb):
    @pl.when(pl.program_id(0) == 0)
    def _init():
        zeros = jnp.zeros_like(sum_ref)
        sum_ref[...] = zeros
        sq_ref[...] = zeros
        mn_ref[...] = jnp.full_like(mn_ref, _BIG)
        mx_ref[...] = jnp.full_like(mx_ref, -_BIG)
        cnt_ref[...] = zeros

    m_scr[...] = jnp.dot(ea_ref[...], wc_ref[...],
                         preferred_element_type=jnp.float32) + bc_ref[...]

    ones = jnp.ones((1, _F), jnp.float32)

    def body(i, _):
        s = src_ref[0, 0, i]
        d = dst_ref[0, 0, i]
        u = b_ref[pl.ds(s, 1), :] + m_scr[pl.ds(i, 1), :]
        dsl = pl.ds(d, 1)
        sum_ref[dsl, :] += u
        sq_ref[dsl, :] += u * u
        mn_ref[dsl, :] = jnp.minimum(mn_ref[dsl, :], u)
        mx_ref[dsl, :] = jnp.maximum(mx_ref[dsl, :], u)
        cnt_ref[dsl, :] += ones
        return 0

    jax.lax.fori_loop(0, eb, body, 0, unroll=32)


def _post_body(x_ref, a_ref, sum_ref, sq_ref, mn_ref, mx_ref, cnt_ref,
               wx_ref, w1_ref, w2_ref, w3_ref, bf_ref, out_ref):
    deg = cnt_ref[:, 0:1]
    has = deg > 0.0
    degc = jnp.maximum(deg, 1.0)
    inv = 1.0 / degc
    a = a_ref[...]
    mean_u = sum_ref[...] * inv
    mean = jnp.where(has, a + mean_u, 0.0)
    mn = jnp.where(has, a + mn_ref[...], 0.0)
    mx = jnp.where(has, a + mx_ref[...], 0.0)
    var = jnp.maximum(sq_ref[...] * inv - mean_u * mean_u, 0.0)
    std = jnp.sqrt(jnp.where(has, var, 0.0) + 1e-5)
    agg = jnp.concatenate([mean, mn, mx, std], axis=-1)
    log_deg = jnp.log(degc + 1.0)
    amp_s = log_deg * (1.0 / _AVG_DEG_LOG)
    att_s = _AVG_DEG_LOG / log_deg
    out = jnp.dot(x_ref[...], wx_ref[...], preferred_element_type=jnp.float32)
    out += jnp.dot(agg, w1_ref[...], preferred_element_type=jnp.float32)
    out += jnp.dot(agg * amp_s, w2_ref[...], preferred_element_type=jnp.float32)
    out += jnp.dot(agg * att_s, w3_ref[...], preferred_element_type=jnp.float32)
    out_ref[...] = out + bf_ref[...]


def kernel(x, edge_index, edge_attr, W_edge, b_edge, W_pre, b_pre, W_post, b_post, W_lin, b_lin):
    n = x.shape[0]
    e = edge_index.shape[1]
    f = _F
    src = edge_index[0]
    dst = edge_index[1]

    wp_d = W_pre[:f]
    wp_s = W_pre[f:2 * f]
    wp_e = W_pre[2 * f:]
    wc = W_edge @ wp_e
    bc = (b_edge @ wp_e + b_pre)[None, :]
    w_pl = W_post @ W_lin
    wx = w_pl[:f]
    w1 = w_pl[f:5 * f]
    w2 = w_pl[5 * f:9 * f]
    w3 = w_pl[9 * f:13 * f]
    bf = (b_post @ W_lin + b_lin)[None, :]

    nb_n = 2000
    a_mat, b_mat = pl.pallas_call(
        _node_ab_body,
        grid=(n // nb_n,),
        in_specs=[
            pl.BlockSpec((nb_n, f), lambda i: (i, 0)),
            pl.BlockSpec((f, f), lambda i: (0, 0)),
            pl.BlockSpec((f, f), lambda i: (0, 0)),
        ],
        out_specs=[
            pl.BlockSpec((nb_n, f), lambda i: (i, 0)),
            pl.BlockSpec((nb_n, f), lambda i: (i, 0)),
        ],
        out_shape=[jax.ShapeDtypeStruct((n, f), jnp.float32)] * 2,
    )(x, wp_d, wp_s)

    eb = 2560
    n_eb = e // eb
    src3 = src.reshape(n_eb, 1, eb)
    dst3 = dst.reshape(n_eb, 1, eb)
    acc_shape = jax.ShapeDtypeStruct((n, f), jnp.float32)
    s_u, s_uu, mn_u, mx_u, cnt = pl.pallas_call(
        functools.partial(_scatter_body, eb=eb),
        grid=(n_eb,),
        in_specs=[
            pl.BlockSpec((1, 1, eb), lambda i: (i, 0, 0), memory_space=pltpu.SMEM),
            pl.BlockSpec((1, 1, eb), lambda i: (i, 0, 0), memory_space=pltpu.SMEM),
            pl.BlockSpec((eb, f), lambda i: (i, 0)),
            pl.BlockSpec((f, f), lambda i: (0, 0)),
            pl.BlockSpec((1, f), lambda i: (0, 0)),
            pl.BlockSpec((n, f), lambda i: (0, 0)),
        ],
        out_specs=[pl.BlockSpec((n, f), lambda i: (0, 0))] * 5,
        out_shape=[acc_shape] * 5,
        scratch_shapes=[pltpu.VMEM((eb, f), jnp.float32)],
    )(src3, dst3, edge_attr, wc, bc, b_mat)

    out = pl.pallas_call(
        _post_body,
        grid=(n // nb_n,),
        in_specs=[
            pl.BlockSpec((nb_n, f), lambda i: (i, 0)),
            pl.BlockSpec((nb_n, f), lambda i: (i, 0)),
            pl.BlockSpec((nb_n, f), lambda i: (i, 0)),
            pl.BlockSpec((nb_n, f), lambda i: (i, 0)),
            pl.BlockSpec((nb_n, f), lambda i: (i, 0)),
            pl.BlockSpec((nb_n, f), lambda i: (i, 0)),
            pl.BlockSpec((nb_n, f), lambda i: (i, 0)),
            pl.BlockSpec((f, f), lambda i: (0, 0)),
            pl.BlockSpec((4 * f, f), lambda i: (0, 0)),
            pl.BlockSpec((4 * f, f), lambda i: (0, 0)),
            pl.BlockSpec((4 * f, f), lambda i: (0, 0)),
            pl.BlockSpec((1, f), lambda i: (0, 0)),
        ],
        out_specs=pl.BlockSpec((nb_n, f), lambda i: (i, 0)),
        out_shape=jax.ShapeDtypeStruct((n, f), jnp.float32),
    )(x, a_mat, s_u, s_uu, mn_u, mx_u, cnt, wx, w1, w2, w3, bf)
    return out
